# baseline (speedup 1.0000x reference)
import functools

import jax
import jax.numpy as jnp
from jax import lax
from jax.experimental import pallas as pl
from jax.experimental.pallas import tpu as pltpu
from jax.experimental.pallas import tpu_sc as plsc

VOCAB = 100000
H = 128
SEQ = 16
BATCH = 16384
NTYPE = 2
NC, NS, L = 2, 16, 16
NW = NC * NS
ROWS = BATCH * SEQ
CB = 16
CHUNK = CB * SEQ
IDROWS = CHUNK // 128
CHUNKS_PER_W = ROWS // (CHUNK * NW)
EPS = 1e-12


def _ln_kernel(ids_hbm, tt_hbm, table_hbm, pos_hbm, typ_hbm, gam_hbm, bet_hbm,
               out_hbm,
               ids_v, tt_v, rows_v, pos_v, typ_v, comb_v, gam_v, bet_v, sem):
    wid = lax.axis_index("c") * NS + lax.axis_index("s")
    iota = lax.iota(jnp.int32, L)

    pltpu.sync_copy(pos_hbm, pos_v)
    pltpu.sync_copy(typ_hbm, typ_v)
    pltpu.sync_copy(gam_hbm, gam_v)
    pltpu.sync_copy(bet_hbm, bet_v)

    def build_comb(s, c):
        for t in range(NTYPE):
            for j in range(H // L):
                comb_v[s * NTYPE + t, pl.ds(j * L, L)] = (
                    pos_v[s, pl.ds(j * L, L)] + typ_v[t, pl.ds(j * L, L)])
        return c
    lax.fori_loop(0, SEQ, build_comb, 0)

    zero = jnp.zeros((L,), jnp.float32)

    def chunk_body(ci, carry):
        t0 = ci * CHUNK
        g0 = ci * IDROWS
        pltpu.sync_copy(ids_hbm.at[pl.ds(g0, IDROWS)], ids_v)
        pltpu.sync_copy(tt_hbm.at[pl.ds(g0, IDROWS)], tt_v)
        cps = [
            pltpu.async_copy(table_hbm.at[ids_v.at[j]],
                             rows_v.at[pl.ds(j * 128, 128)], sem)
            for j in range(IDROWS)
        ]
        for cp in cps:
            cp.wait()

        for bl in range(CB):
            r, cc = bl // 8, bl % 8
            row_idx = iota + bl * SEQ
            ttv = tt_v[r, pl.ds(cc * L, L)]
            crow = iota * NTYPE + ttv

            def pass1(h, carry):
                s1, s2 = carry
                hv = jnp.broadcast_to(h, (L,)).astype(jnp.int32)
                v = plsc.load_gather(rows_v, [row_idx, hv])
                v = v + plsc.load_gather(comb_v, [crow, hv])
                plsc.store_scatter(rows_v, [row_idx, hv], v)
                return (s1 + v, s2 + v * v)

            s1, s2 = lax.fori_loop(0, H, pass1, (zero, zero))
            mean = s1 * (1.0 / H)
            var = s2 * (1.0 / H) - mean * mean
            x = var + EPS
            i = plsc.bitcast(x, jnp.int32)
            y = plsc.bitcast(jnp.int32(0x5F3759DF) - (i >> 1), jnp.float32)
            for _ in range(3):
                y = y * (1.5 - 0.5 * x * y * y)
            rstd = y

            def pass2(h, c):
                hv = jnp.broadcast_to(h, (L,)).astype(jnp.int32)
                v = plsc.load_gather(rows_v, [row_idx, hv])
                g = plsc.load_gather(gam_v, [hv])
                b = plsc.load_gather(bet_v, [hv])
                o = (v - mean) * rstd * g + b
                plsc.store_scatter(rows_v, [row_idx, hv], o)
                return c

            lax.fori_loop(0, H, pass2, 0)

        pltpu.sync_copy(rows_v, out_hbm.at[pl.ds(t0, CHUNK)])
        return carry

    lax.fori_loop(wid * CHUNKS_PER_W, (wid + 1) * CHUNKS_PER_W, chunk_body, 0)


_ln_call = functools.partial(
    pl.kernel,
    out_type=jax.ShapeDtypeStruct((ROWS, H), jnp.float32),
    mesh=plsc.VectorSubcoreMesh(core_axis_name="c", subcore_axis_name="s"),
    compiler_params=pltpu.CompilerParams(needs_layout_passes=False),
    scratch_types=[
        pltpu.VMEM((IDROWS, 128), jnp.int32),
        pltpu.VMEM((IDROWS, 128), jnp.int32),
        pltpu.VMEM((CHUNK, H), jnp.float32),
        pltpu.VMEM((SEQ, H), jnp.float32),
        pltpu.VMEM((NTYPE, H), jnp.float32),
        pltpu.VMEM((SEQ * NTYPE, H), jnp.float32),
        pltpu.VMEM((H,), jnp.float32),
        pltpu.VMEM((H,), jnp.float32),
        pltpu.SemaphoreType.DMA,
    ],
)(_ln_kernel)


def kernel(input_ids, token_type_ids, pokemon_table, position_table, type_table,
           gamma, beta):
    ids2 = input_ids.reshape(ROWS // 128, 128).astype(jnp.int32)
    tt2 = token_type_ids.reshape(ROWS // 128, 128).astype(jnp.int32)
    out = _ln_call(ids2, tt2, pokemon_table, position_table, type_table,
                   gamma, beta)
    return out.reshape(BATCH, SEQ, H)


# speedup vs baseline: 1.7184x; 1.7184x over previous
import functools

import jax
import jax.numpy as jnp
from jax import lax
from jax.experimental import pallas as pl
from jax.experimental.pallas import tpu as pltpu
from jax.experimental.pallas import tpu_sc as plsc

VOCAB = 100000
H = 128
SEQ = 16
BATCH = 16384
NTYPE = 2
NC, NS, L = 2, 16, 16
NW = NC * NS
ROWS = BATCH * SEQ
CB = 16
CHUNK = CB * SEQ
IDROWS = CHUNK // 128
CHUNKS_PER_W = ROWS // (CHUNK * NW)
EPS = 1e-12


def _ln_kernel(ids_hbm, tt_hbm, table_hbm, pos_hbm, typ_hbm, gam_hbm, bet_hbm,
               out_hbm,
               ids_v, tt_v, rows_v, pos_v, typ_v, comb_v, gam_v, bet_v, sem):
    wid = lax.axis_index("c") * NS + lax.axis_index("s")
    iota = lax.iota(jnp.int32, L)

    pltpu.sync_copy(pos_hbm, pos_v)
    pltpu.sync_copy(typ_hbm, typ_v)
    pltpu.sync_copy(gam_hbm, gam_v)
    pltpu.sync_copy(bet_hbm, bet_v)

    def build_comb(s, c):
        for t in range(NTYPE):
            for j in range(H // L):
                comb_v[s * NTYPE + t, pl.ds(j * L, L)] = (
                    pos_v[s, pl.ds(j * L, L)] + typ_v[t, pl.ds(j * L, L)])
        return c
    lax.fori_loop(0, SEQ, build_comb, 0)

    zero = jnp.zeros((L,), jnp.float32)

    def chunk_body(ci, carry):
        t0 = ci * CHUNK
        g0 = ci * IDROWS
        pltpu.sync_copy(ids_hbm.at[pl.ds(g0, IDROWS)], ids_v)
        pltpu.sync_copy(tt_hbm.at[pl.ds(g0, IDROWS)], tt_v)
        cps = [
            pltpu.async_copy(table_hbm.at[ids_v.at[j]],
                             rows_v.at[pl.ds(j * 128, 128)], sem)
            for j in range(IDROWS)
        ]
        for cp in cps:
            cp.wait()

        for bl in range(CB):
            r, cc = bl // 8, bl % 8
            row_idx = iota + bl * SEQ
            ttv = tt_v[r, pl.ds(cc * L, L)]
            crow = iota * NTYPE + ttv

            @plsc.parallel_loop(0, H, 1, unroll=8, carry=(zero, zero))
            def stats(h, carry):
                s1, s2 = carry
                hv = jnp.broadcast_to(h, (L,)).astype(jnp.int32)
                v = plsc.load_gather(rows_v, [row_idx, hv])
                v = v + plsc.load_gather(comb_v, [crow, hv])
                plsc.store_scatter(rows_v, [row_idx, hv], v)
                return (s1 + v, s2 + v * v)

            s1, s2 = stats
            mean = s1 * (1.0 / H)
            var = s2 * (1.0 / H) - mean * mean
            x = var + EPS
            i = plsc.bitcast(x, jnp.int32)
            y = plsc.bitcast(jnp.int32(0x5F3759DF) - (i >> 1), jnp.float32)
            for _ in range(3):
                y = y * (1.5 - 0.5 * x * y * y)
            rstd = y

            @plsc.parallel_loop(0, H, 1, unroll=8)
            def normalize(h):
                hv = jnp.broadcast_to(h, (L,)).astype(jnp.int32)
                v = plsc.load_gather(rows_v, [row_idx, hv])
                g = plsc.load_gather(gam_v, [hv])
                b = plsc.load_gather(bet_v, [hv])
                o = (v - mean) * rstd * g + b
                plsc.store_scatter(rows_v, [row_idx, hv], o)

        pltpu.sync_copy(rows_v, out_hbm.at[pl.ds(t0, CHUNK)])
        return carry

    lax.fori_loop(wid * CHUNKS_PER_W, (wid + 1) * CHUNKS_PER_W, chunk_body, 0)


_ln_call = functools.partial(
    pl.kernel,
    out_type=jax.ShapeDtypeStruct((ROWS, H), jnp.float32),
    mesh=plsc.VectorSubcoreMesh(core_axis_name="c", subcore_axis_name="s"),
    compiler_params=pltpu.CompilerParams(needs_layout_passes=False),
    scratch_types=[
        pltpu.VMEM((IDROWS, 128), jnp.int32),
        pltpu.VMEM((IDROWS, 128), jnp.int32),
        pltpu.VMEM((CHUNK, H), jnp.float32),
        pltpu.VMEM((SEQ, H), jnp.float32),
        pltpu.VMEM((NTYPE, H), jnp.float32),
        pltpu.VMEM((SEQ * NTYPE, H), jnp.float32),
        pltpu.VMEM((H,), jnp.float32),
        pltpu.VMEM((H,), jnp.float32),
        pltpu.SemaphoreType.DMA,
    ],
)(_ln_kernel)


def kernel(input_ids, token_type_ids, pokemon_table, position_table, type_table,
           gamma, beta):
    ids2 = input_ids.reshape(ROWS // 128, 128).astype(jnp.int32)
    tt2 = token_type_ids.reshape(ROWS // 128, 128).astype(jnp.int32)
    out = _ln_call(ids2, tt2, pokemon_table, position_table, type_table,
                   gamma, beta)
    return out.reshape(BATCH, SEQ, H)
